# Initial kernel scaffold; baseline (speedup 1.0000x reference)
import jax
import jax.numpy as jnp
from jax.experimental import pallas as pl


def kernel(x, edge_index, edge_attr, batch, W1, b1, W2, b2, W3, b3, W4, b4, W5, b5, Wfc, bfc):
    raise NotImplementedError("write your pallas kernel here")


# speedup vs baseline: 8.6250x; 8.6250x over previous
import functools

import jax
import jax.numpy as jnp
from jax import lax
from jax.experimental import pallas as pl
from jax.experimental.pallas import tpu as pltpu
from jax.experimental.pallas import tpu_sc as plsc

N, E, FIN, H, C, G = 10000, 320000, 128, 64, 10, 64
NC, NS = 2, 16
NW = NC * NS
EPW = E // NW
CH = 80
NCHUNK = EPW // CH
NPAD = 10240
RPT = NPAD // NS
MB = 2048
NMB = NPAD // MB

@functools.cache
def _sc_kernels():
    mesh = plsc.VectorSubcoreMesh(
        core_axis_name="c", subcore_axis_name="s", num_cores=NC,
        num_subcores=NS,
    )
    params = pltpu.CompilerParams(use_tc_tiling_on_sc=False)
    deg = functools.partial(
        pl.kernel,
        out_type=jax.ShapeDtypeStruct((NC, NPAD), jnp.float32),
        mesh=mesh,
        compiler_params=params,
        scratch_types=[
            pltpu.VMEM((CH,), jnp.int32),
            pltpu.VMEM((CH,), jnp.float32),
            pltpu.VMEM((RPT,), jnp.float32),
            pltpu.VMEM_SHARED((NPAD,), jnp.float32),
            pltpu.SemaphoreType.DMA,
        ],
    )(_deg_body)
    agg = functools.partial(
        pl.kernel,
        out_type=jax.ShapeDtypeStruct((NC, NPAD, H), jnp.float32),
        mesh=mesh,
        compiler_params=params,
        scratch_types=[
            pltpu.VMEM((CH,), jnp.int32),
            pltpu.VMEM((CH,), jnp.int32),
            pltpu.VMEM((CH,), jnp.float32),
            pltpu.VMEM((CH, H), jnp.float32),
            pltpu.VMEM((RPT, H), jnp.float32),
            pltpu.VMEM_SHARED((NPAD, H), jnp.float32),
            pltpu.SemaphoreType.DMA,
        ],
    )(_agg_body)
    return deg, agg


def _deg_body(col_hbm, ew_hbm, out_hbm, colbuf, ewbuf, zbuf, acc, sem):
    del sem
    cid = lax.axis_index("c")
    sid = lax.axis_index("s")
    wid = sid * NC + cid
    base = wid * EPW

    def zb(i, _):
        zbuf[pl.ds(i * 16, 16)] = jnp.zeros((16,), jnp.float32)
        return 0

    lax.fori_loop(0, RPT // 16, zb, 0)
    pltpu.sync_copy(zbuf, acc.at[pl.ds(sid * RPT, RPT)])
    plsc.subcore_barrier()

    def chunk(k, _):
        off = base + k * CH
        pltpu.sync_copy(col_hbm.at[pl.ds(off, CH)], colbuf)
        pltpu.sync_copy(ew_hbm.at[pl.ds(off, CH)], ewbuf)
        pltpu.sync_copy(ewbuf, acc.at[colbuf], add=True)
        return 0

    lax.fori_loop(0, NCHUNK, chunk, 0)
    plsc.subcore_barrier()
    pltpu.sync_copy(
        acc.at[pl.ds(sid * RPT, RPT)], out_hbm.at[cid, pl.ds(sid * RPT, RPT)]
    )


def _agg_body(
    hs_hbm, row_hbm, col_hbm, ew_hbm, out_hbm,
    rowbuf, colbuf, ewbuf, rows, zbuf, acc, sem,
):
    cid = lax.axis_index("c")
    sid = lax.axis_index("s")
    wid = sid * NC + cid
    base = wid * EPW

    def zb(i, _):
        for q in range(H // 16):
            zbuf[i, pl.ds(q * 16, 16)] = jnp.zeros((16,), jnp.float32)
        return 0

    lax.fori_loop(0, RPT, zb, 0)
    pltpu.sync_copy(zbuf, acc.at[pl.ds(sid * RPT, RPT)])
    plsc.subcore_barrier()

    def chunk(k, _):
        off = base + k * CH
        pltpu.sync_copy(row_hbm.at[pl.ds(off, CH)], rowbuf)
        pltpu.sync_copy(col_hbm.at[pl.ds(off, CH)], colbuf)
        pltpu.sync_copy(ew_hbm.at[pl.ds(off, CH)], ewbuf)
        pltpu.async_copy(hs_hbm.at[rowbuf], rows, sem).wait()
        for g in range(CH // 16):
            ewv = ewbuf[pl.ds(g * 16, 16)]
            for j in range(16):
                w = ewv.at[jnp.full((16,), j, jnp.int32)].get(
                    mode="promise_in_bounds"
                )
                e = g * 16 + j
                for q in range(H // 16):
                    sl = pl.ds(q * 16, 16)
                    rows[e, sl] = rows[e, sl] * w
        pltpu.sync_copy(rows, acc.at[colbuf], add=True)
        return 0

    lax.fori_loop(0, NCHUNK, chunk, 0)
    plsc.subcore_barrier()
    pltpu.sync_copy(
        acc.at[pl.ds(sid * RPT, RPT)], out_hbm.at[cid, pl.ds(sid * RPT, RPT)]
    )


def _dinv_block(degpt):
    deg = degpt[:, 0:1] + degpt[:, 1:2] + 1.0
    return lax.rsqrt(deg)


def _mm1_body(x_ref, w_ref, degpt_ref, o_ref):
    dinv = _dinv_block(degpt_ref[...])
    h = jnp.dot(x_ref[...], w_ref[...], preferred_element_type=jnp.float32)
    o_ref[...] = h * dinv


def _layer_body(aggp_ref, hs_ref, degpt_ref, b_ref, w_ref, o_ref):
    dinv = _dinv_block(degpt_ref[...])
    s = aggp_ref[0] + aggp_ref[1] + hs_ref[...]
    act = jnp.maximum(s * dinv + b_ref[...], 0.0)
    o_ref[...] = (
        jnp.dot(act, w_ref[...], preferred_element_type=jnp.float32) * dinv
    )


def _pool_body(
    aggp_ref, hs_ref, degpt_ref, b_ref, batch_ref, wfc_ref, bfc_ref,
    o_ref, acc_ref, cnt_ref,
):
    i = pl.program_id(0)

    @pl.when(i == 0)
    def _():
        acc_ref[...] = jnp.zeros_like(acc_ref)
        cnt_ref[...] = jnp.zeros_like(cnt_ref)

    dinv = _dinv_block(degpt_ref[...])
    s = aggp_ref[0] + aggp_ref[1] + hs_ref[...]
    act = jnp.maximum(s * dinv + b_ref[...], 0.0)
    bvals = batch_ref[0]
    iot = lax.broadcasted_iota(jnp.int32, (G, MB), 0)
    oh = (iot == bvals).astype(jnp.float32)
    acc_ref[...] += jnp.dot(oh, act, preferred_element_type=jnp.float32)
    cnt_ref[...] += jnp.sum(oh, axis=1, keepdims=True)
    pooled = acc_ref[...] / jnp.maximum(cnt_ref[...], 1.0)
    o_ref[...] = (
        jnp.dot(pooled, wfc_ref[...], preferred_element_type=jnp.float32)
        + bfc_ref[...]
    )


_mm1 = pl.pallas_call(
    _mm1_body,
    grid=(NMB,),
    in_specs=[
        pl.BlockSpec((MB, FIN), lambda i: (i, 0)),
        pl.BlockSpec((FIN, H), lambda i: (0, 0)),
        pl.BlockSpec((MB, NC), lambda i: (i, 0)),
    ],
    out_specs=pl.BlockSpec((MB, H), lambda i: (i, 0)),
    out_shape=jax.ShapeDtypeStruct((NPAD, H), jnp.float32),
)

_layer = pl.pallas_call(
    _layer_body,
    grid=(NMB,),
    in_specs=[
        pl.BlockSpec((NC, MB, H), lambda i: (0, i, 0)),
        pl.BlockSpec((MB, H), lambda i: (i, 0)),
        pl.BlockSpec((MB, NC), lambda i: (i, 0)),
        pl.BlockSpec((1, H), lambda i: (0, 0)),
        pl.BlockSpec((H, H), lambda i: (0, 0)),
    ],
    out_specs=pl.BlockSpec((MB, H), lambda i: (i, 0)),
    out_shape=jax.ShapeDtypeStruct((NPAD, H), jnp.float32),
)

_pool = pl.pallas_call(
    _pool_body,
    grid=(NMB,),
    in_specs=[
        pl.BlockSpec((NC, MB, H), lambda i: (0, i, 0)),
        pl.BlockSpec((MB, H), lambda i: (i, 0)),
        pl.BlockSpec((MB, NC), lambda i: (i, 0)),
        pl.BlockSpec((1, H), lambda i: (0, 0)),
        pl.BlockSpec((1, 1, MB), lambda i: (i, 0, 0)),
        pl.BlockSpec((H, C), lambda i: (0, 0)),
        pl.BlockSpec((1, C), lambda i: (0, 0)),
    ],
    out_specs=pl.BlockSpec((G, C), lambda i: (0, 0)),
    out_shape=jax.ShapeDtypeStruct((G, C), jnp.float32),
    scratch_shapes=[
        pltpu.VMEM((G, H), jnp.float32),
        pltpu.VMEM((G, 1), jnp.float32),
    ],
)


def kernel(x, edge_index, edge_attr, batch, W1, b1, W2, b2, W3, b3, W4, b4,
           W5, b5, Wfc, bfc):
    row, col = edge_index[0], edge_index[1]
    _deg_kernel, _agg_kernel = _sc_kernels()
    degp = _deg_kernel(col, edge_attr)
    degpt = degp.T
    xp = jnp.concatenate(
        [x, jnp.zeros((NPAD - N, FIN), jnp.float32)], axis=0
    )
    bpad = jnp.concatenate(
        [batch, jnp.full((NPAD - N,), G, batch.dtype)]
    ).reshape(NMB, 1, MB)

    hs = _mm1(xp, W1, degpt)
    for b_prev, W_next in ((b1, W2), (b2, W3), (b3, W4), (b4, W5)):
        aggp = _agg_kernel(hs, row, col, edge_attr)
        hs = _layer(aggp, hs, degpt, b_prev.reshape(1, H), W_next)
    aggp = _agg_kernel(hs, row, col, edge_attr)
    return _pool(
        aggp, hs, degpt, b5.reshape(1, H), bpad, Wfc, bfc.reshape(1, C)
    )


# speedup vs baseline: 19.0515x; 2.2089x over previous
import functools

import jax
import jax.numpy as jnp
from jax import lax
from jax.experimental import pallas as pl
from jax.experimental.pallas import tpu as pltpu
from jax.experimental.pallas import tpu_sc as plsc

N, E, FIN, H, C, G = 10000, 320000, 128, 64, 10, 64
NC, NS = 2, 16
NW = NC * NS
EPW = E // NW
CH = 80
NCHUNK = EPW // CH
NPAD = 10240
RPT = NPAD // NS
MB = 2048
NMB = NPAD // MB

@functools.cache
def _sc_kernels():
    mesh = plsc.VectorSubcoreMesh(
        core_axis_name="c", subcore_axis_name="s", num_cores=NC,
        num_subcores=NS,
    )
    params = pltpu.CompilerParams(use_tc_tiling_on_sc=False)
    deg = functools.partial(
        pl.kernel,
        out_type=jax.ShapeDtypeStruct((NC, NPAD), jnp.float32),
        mesh=mesh,
        compiler_params=params,
        scratch_types=[
            pltpu.VMEM((CH,), jnp.int32),
            pltpu.VMEM((CH,), jnp.float32),
            pltpu.VMEM((RPT,), jnp.float32),
            pltpu.VMEM_SHARED((NPAD,), jnp.float32),
            pltpu.SemaphoreType.DMA,
        ],
    )(_deg_body)
    agg = functools.partial(
        pl.kernel,
        out_type=jax.ShapeDtypeStruct((NC, NPAD, H), jnp.float32),
        mesh=mesh,
        compiler_params=params,
        scratch_types=[
            pltpu.VMEM((3, CH), jnp.int32),
            pltpu.VMEM((3, CH), jnp.int32),
            pltpu.VMEM((3, CH), jnp.float32),
            pltpu.VMEM((2, CH, H), jnp.float32),
            pltpu.VMEM((RPT, H), jnp.float32),
            pltpu.VMEM_SHARED((NPAD, H), jnp.float32),
            pltpu.SemaphoreType.DMA,
            pltpu.SemaphoreType.DMA,
            pltpu.SemaphoreType.DMA,
        ],
    )(_agg_body)
    return deg, agg


def _deg_body(col_hbm, ew_hbm, out_hbm, colbuf, ewbuf, zbuf, acc, sem):
    del sem
    cid = lax.axis_index("c")
    sid = lax.axis_index("s")
    wid = sid * NC + cid
    base = wid * EPW

    def zb(i, _):
        zbuf[pl.ds(i * 16, 16)] = jnp.zeros((16,), jnp.float32)
        return 0

    lax.fori_loop(0, RPT // 16, zb, 0)
    pltpu.sync_copy(zbuf, acc.at[pl.ds(sid * RPT, RPT)])
    plsc.subcore_barrier()

    def chunk(k, _):
        off = base + k * CH
        pltpu.sync_copy(col_hbm.at[pl.ds(off, CH)], colbuf)
        pltpu.sync_copy(ew_hbm.at[pl.ds(off, CH)], ewbuf)
        pltpu.sync_copy(ewbuf, acc.at[colbuf], add=True)
        return 0

    lax.fori_loop(0, NCHUNK, chunk, 0)
    plsc.subcore_barrier()
    pltpu.sync_copy(
        acc.at[pl.ds(sid * RPT, RPT)], out_hbm.at[cid, pl.ds(sid * RPT, RPT)]
    )


def _agg_body(
    hs_hbm, row_hbm, col_hbm, ew_hbm, out_hbm,
    rowb, colb, ewb, rows, zbuf, acc, isem, gsem, ssem,
):
    cid = lax.axis_index("c")
    sid = lax.axis_index("s")
    wid = sid * NC + cid
    base = wid * EPW

    def zb(i, _):
        for q in range(H // 16):
            zbuf[i, pl.ds(q * 16, 16)] = jnp.zeros((16,), jnp.float32)
        return 0

    lax.fori_loop(0, RPT, zb, 0)
    pltpu.sync_copy(zbuf, acc.at[pl.ds(sid * RPT, RPT)])
    plsc.subcore_barrier()

    def issue_idx(k, slot):
        off = base + k * CH
        pltpu.async_copy(row_hbm.at[pl.ds(off, CH)], rowb.at[slot], isem)
        pltpu.async_copy(col_hbm.at[pl.ds(off, CH)], colb.at[slot], isem)
        pltpu.async_copy(ew_hbm.at[pl.ds(off, CH)], ewb.at[slot], isem)

    def drain_idx(slot):
        pltpu.make_async_copy(
            row_hbm.at[pl.ds(base, CH)], rowb.at[slot], isem
        ).wait()
        pltpu.make_async_copy(
            col_hbm.at[pl.ds(base, CH)], colb.at[slot], isem
        ).wait()
        pltpu.make_async_copy(
            ew_hbm.at[pl.ds(base, CH)], ewb.at[slot], isem
        ).wait()

    pltpu.sync_copy(row_hbm.at[pl.ds(base, CH)], rowb.at[0])
    pltpu.sync_copy(col_hbm.at[pl.ds(base, CH)], colb.at[0])
    pltpu.sync_copy(ew_hbm.at[pl.ds(base, CH)], ewb.at[0])
    pltpu.async_copy(hs_hbm.at[rowb.at[0]], rows.at[0], gsem)
    issue_idx(1, 1)

    def chunk(k, _):
        b = lax.rem(k, 2)
        nb = 1 - b
        islot = lax.rem(k, 3)
        inext = lax.rem(k + 1, 3)
        iprev = lax.rem(k + 2, 3)
        @pl.when(k > 0)
        def _():
            pltpu.make_async_copy(
                rows.at[nb], acc.at[colb.at[iprev]], ssem
            ).wait()

        @pl.when(k < NCHUNK - 1)
        def _():
            drain_idx(inext)

        pltpu.make_async_copy(hs_hbm.at[rowb.at[islot]], rows.at[b], gsem).wait()

        @pl.when(k < NCHUNK - 2)
        def _():
            issue_idx(k + 2, iprev)

        @pl.when(k < NCHUNK - 1)
        def _():
            pltpu.async_copy(hs_hbm.at[rowb.at[inext]], rows.at[nb], gsem)

        for g in range(CH // 16):
            ewv = ewb[islot, pl.ds(g * 16, 16)]
            for j in range(16):
                w = ewv.at[jnp.full((16,), j, jnp.int32)].get(
                    mode="promise_in_bounds"
                )
                e = g * 16 + j
                for q in range(H // 16):
                    sl = pl.ds(q * 16, 16)
                    rows[b, e, sl] = rows[b, e, sl] * w

        pltpu.async_copy(rows.at[b], acc.at[colb.at[islot]], ssem, add=True)
        return 0

    lax.fori_loop(0, NCHUNK, chunk, 0)
    lastb = (NCHUNK - 1) % 2
    lasts = (NCHUNK - 1) % 3
    pltpu.make_async_copy(
        rows.at[lastb], acc.at[colb.at[lasts]], ssem
    ).wait()
    plsc.subcore_barrier()
    pltpu.sync_copy(
        acc.at[pl.ds(sid * RPT, RPT)], out_hbm.at[cid, pl.ds(sid * RPT, RPT)]
    )


def _dinv_block(degpt):
    deg = degpt[:, 0:1] + degpt[:, 1:2] + 1.0
    return lax.rsqrt(deg)


def _mm1_body(x_ref, w_ref, degpt_ref, o_ref):
    dinv = _dinv_block(degpt_ref[...])
    h = jnp.dot(x_ref[...], w_ref[...], preferred_element_type=jnp.float32)
    o_ref[...] = h * dinv


def _layer_body(aggp_ref, hs_ref, degpt_ref, b_ref, w_ref, o_ref):
    dinv = _dinv_block(degpt_ref[...])
    s = aggp_ref[0] + aggp_ref[1] + hs_ref[...]
    act = jnp.maximum(s * dinv + b_ref[...], 0.0)
    o_ref[...] = (
        jnp.dot(act, w_ref[...], preferred_element_type=jnp.float32) * dinv
    )


def _pool_body(
    aggp_ref, hs_ref, degpt_ref, b_ref, batch_ref, wfc_ref, bfc_ref,
    o_ref, acc_ref, cnt_ref,
):
    i = pl.program_id(0)

    @pl.when(i == 0)
    def _():
        acc_ref[...] = jnp.zeros_like(acc_ref)
        cnt_ref[...] = jnp.zeros_like(cnt_ref)

    dinv = _dinv_block(degpt_ref[...])
    s = aggp_ref[0] + aggp_ref[1] + hs_ref[...]
    act = jnp.maximum(s * dinv + b_ref[...], 0.0)
    bvals = batch_ref[0]
    iot = lax.broadcasted_iota(jnp.int32, (G, MB), 0)
    oh = (iot == bvals).astype(jnp.float32)
    acc_ref[...] += jnp.dot(oh, act, preferred_element_type=jnp.float32)
    cnt_ref[...] += jnp.sum(oh, axis=1, keepdims=True)
    pooled = acc_ref[...] / jnp.maximum(cnt_ref[...], 1.0)
    o_ref[...] = (
        jnp.dot(pooled, wfc_ref[...], preferred_element_type=jnp.float32)
        + bfc_ref[...]
    )


_mm1 = pl.pallas_call(
    _mm1_body,
    grid=(NMB,),
    in_specs=[
        pl.BlockSpec((MB, FIN), lambda i: (i, 0)),
        pl.BlockSpec((FIN, H), lambda i: (0, 0)),
        pl.BlockSpec((MB, NC), lambda i: (i, 0)),
    ],
    out_specs=pl.BlockSpec((MB, H), lambda i: (i, 0)),
    out_shape=jax.ShapeDtypeStruct((NPAD, H), jnp.float32),
)

_layer = pl.pallas_call(
    _layer_body,
    grid=(NMB,),
    in_specs=[
        pl.BlockSpec((NC, MB, H), lambda i: (0, i, 0)),
        pl.BlockSpec((MB, H), lambda i: (i, 0)),
        pl.BlockSpec((MB, NC), lambda i: (i, 0)),
        pl.BlockSpec((1, H), lambda i: (0, 0)),
        pl.BlockSpec((H, H), lambda i: (0, 0)),
    ],
    out_specs=pl.BlockSpec((MB, H), lambda i: (i, 0)),
    out_shape=jax.ShapeDtypeStruct((NPAD, H), jnp.float32),
)

_pool = pl.pallas_call(
    _pool_body,
    grid=(NMB,),
    in_specs=[
        pl.BlockSpec((NC, MB, H), lambda i: (0, i, 0)),
        pl.BlockSpec((MB, H), lambda i: (i, 0)),
        pl.BlockSpec((MB, NC), lambda i: (i, 0)),
        pl.BlockSpec((1, H), lambda i: (0, 0)),
        pl.BlockSpec((1, 1, MB), lambda i: (i, 0, 0)),
        pl.BlockSpec((H, C), lambda i: (0, 0)),
        pl.BlockSpec((1, C), lambda i: (0, 0)),
    ],
    out_specs=pl.BlockSpec((G, C), lambda i: (0, 0)),
    out_shape=jax.ShapeDtypeStruct((G, C), jnp.float32),
    scratch_shapes=[
        pltpu.VMEM((G, H), jnp.float32),
        pltpu.VMEM((G, 1), jnp.float32),
    ],
)


def kernel(x, edge_index, edge_attr, batch, W1, b1, W2, b2, W3, b3, W4, b4,
           W5, b5, Wfc, bfc):
    row, col = edge_index[0], edge_index[1]
    _deg_kernel, _agg_kernel = _sc_kernels()
    degp = _deg_kernel(col, edge_attr)
    degpt = degp.T
    xp = jnp.concatenate(
        [x, jnp.zeros((NPAD - N, FIN), jnp.float32)], axis=0
    )
    bpad = jnp.concatenate(
        [batch, jnp.full((NPAD - N,), G, batch.dtype)]
    ).reshape(NMB, 1, MB)

    hs = _mm1(xp, W1, degpt)
    for b_prev, W_next in ((b1, W2), (b2, W3), (b3, W4), (b4, W5)):
        aggp = _agg_kernel(hs, row, col, edge_attr)
        hs = _layer(aggp, hs, degpt, b_prev.reshape(1, H), W_next)
    aggp = _agg_kernel(hs, row, col, edge_attr)
    return _pool(
        aggp, hs, degpt, b5.reshape(1, H), bpad, Wfc, bfc.reshape(1, C)
    )


# speedup vs baseline: 20.0035x; 1.0500x over previous
import functools

import jax
import jax.numpy as jnp
from jax import lax
from jax.experimental import pallas as pl
from jax.experimental.pallas import tpu as pltpu
from jax.experimental.pallas import tpu_sc as plsc

N, E, FIN, H, C, G = 10000, 320000, 128, 64, 10, 64
NC, NS = 2, 16
NW = NC * NS
EPW = E // NW
CH = 80
SUBS = 5
SUP = CH * SUBS
NSUP = EPW // SUP
NCHUNK = EPW // CH
NPAD = 10240
RPT = NPAD // NS
MB = 2048
NMB = NPAD // MB

@functools.cache
def _sc_kernels():
    mesh = plsc.VectorSubcoreMesh(
        core_axis_name="c", subcore_axis_name="s", num_cores=NC,
        num_subcores=NS,
    )
    params = pltpu.CompilerParams(use_tc_tiling_on_sc=False)
    deg = functools.partial(
        pl.kernel,
        out_type=jax.ShapeDtypeStruct((NC, NPAD), jnp.float32),
        mesh=mesh,
        compiler_params=params,
        scratch_types=[
            pltpu.VMEM((CH,), jnp.int32),
            pltpu.VMEM((CH,), jnp.float32),
            pltpu.VMEM((RPT,), jnp.float32),
            pltpu.VMEM_SHARED((NPAD,), jnp.float32),
            pltpu.SemaphoreType.DMA,
        ],
    )(_deg_body)
    agg = functools.partial(
        pl.kernel,
        out_type=jax.ShapeDtypeStruct((NC, NPAD, H), jnp.float32),
        mesh=mesh,
        compiler_params=params,
        scratch_types=[
            pltpu.VMEM((3, SUBS, CH), jnp.int32),
            pltpu.VMEM((3, SUBS, CH), jnp.int32),
            pltpu.VMEM((3, SUBS, CH), jnp.float32),
            pltpu.VMEM((2, SUBS, CH, H), jnp.float32),
            pltpu.VMEM_SHARED((NPAD, H), jnp.float32),
            pltpu.SemaphoreType.DMA,
            pltpu.SemaphoreType.DMA,
            pltpu.SemaphoreType.DMA,
        ],
    )(_agg_body)
    return deg, agg


def _deg_body(col_hbm, ew_hbm, out_hbm, colbuf, ewbuf, zbuf, acc, sem):
    del sem
    cid = lax.axis_index("c")
    sid = lax.axis_index("s")
    wid = sid * NC + cid
    base = wid * EPW

    def zb(i, _):
        zbuf[pl.ds(i * 16, 16)] = jnp.zeros((16,), jnp.float32)
        return 0

    lax.fori_loop(0, RPT // 16, zb, 0)
    pltpu.sync_copy(zbuf, acc.at[pl.ds(sid * RPT, RPT)])
    plsc.subcore_barrier()

    def chunk(k, _):
        off = base + k * CH
        pltpu.sync_copy(col_hbm.at[pl.ds(off, CH)], colbuf)
        pltpu.sync_copy(ew_hbm.at[pl.ds(off, CH)], ewbuf)
        pltpu.sync_copy(ewbuf, acc.at[colbuf], add=True)
        return 0

    lax.fori_loop(0, NCHUNK, chunk, 0)
    plsc.subcore_barrier()
    pltpu.sync_copy(
        acc.at[pl.ds(sid * RPT, RPT)], out_hbm.at[cid, pl.ds(sid * RPT, RPT)]
    )


def _agg_body(
    hs_hbm, row_hbm, col_hbm, ew_hbm, zeros_hbm, out_hbm,
    rowb, colb, ewb, rows, acc, isem, gsem, ssem,
):
    cid = lax.axis_index("c")
    sid = lax.axis_index("s")
    wid = sid * NC + cid
    base = wid * (EPW // CH)

    pltpu.sync_copy(zeros_hbm, acc.at[pl.ds(sid * RPT, RPT)])
    plsc.subcore_barrier()

    def issue_idx(k, slot):
        off = base + k * SUBS
        pltpu.async_copy(row_hbm.at[pl.ds(off, SUBS)], rowb.at[slot], isem)
        pltpu.async_copy(col_hbm.at[pl.ds(off, SUBS)], colb.at[slot], isem)
        pltpu.async_copy(ew_hbm.at[pl.ds(off, SUBS)], ewb.at[slot], isem)

    def drain_idx(slot):
        pltpu.make_async_copy(
            row_hbm.at[pl.ds(base, SUBS)], rowb.at[slot], isem
        ).wait()
        pltpu.make_async_copy(
            col_hbm.at[pl.ds(base, SUBS)], colb.at[slot], isem
        ).wait()
        pltpu.make_async_copy(
            ew_hbm.at[pl.ds(base, SUBS)], ewb.at[slot], isem
        ).wait()

    def issue_gathers(slot, b):
        for j in range(SUBS):
            pltpu.async_copy(
                hs_hbm.at[rowb.at[slot, j]], rows.at[b, j], gsem
            )

    def drain_gathers(slot, b):
        for j in range(SUBS):
            pltpu.make_async_copy(
                hs_hbm.at[rowb.at[slot, j]], rows.at[b, j], gsem
            ).wait()

    pltpu.sync_copy(row_hbm.at[pl.ds(base, SUBS)], rowb.at[0])
    pltpu.sync_copy(col_hbm.at[pl.ds(base, SUBS)], colb.at[0])
    pltpu.sync_copy(ew_hbm.at[pl.ds(base, SUBS)], ewb.at[0])
    issue_gathers(0, 0)
    issue_idx(1, 1)

    def chunk(k, _):
        b = lax.rem(k, 2)
        nb = 1 - b
        islot = lax.rem(k, 3)
        inext = lax.rem(k + 1, 3)
        iprev = lax.rem(k + 2, 3)
        @pl.when(k > 0)
        def _():
            for j in range(SUBS):
                pltpu.make_async_copy(
                    rows.at[nb, j], acc.at[colb.at[iprev, j]], ssem
                ).wait()

        @pl.when(k < NSUP - 1)
        def _():
            drain_idx(inext)

        drain_gathers(islot, b)

        @pl.when(k < NSUP - 2)
        def _():
            issue_idx(k + 2, iprev)

        @pl.when(k < NSUP - 1)
        def _():
            issue_gathers(inext, nb)

        for j in range(SUBS):
            for g in range(CH // 16):
                ewv = ewb[islot, j, pl.ds(g * 16, 16)]
                for l in range(16):
                    w = ewv.at[jnp.full((16,), l, jnp.int32)].get(
                        mode="promise_in_bounds"
                    )
                    e = g * 16 + l
                    for q in range(H // 16):
                        sl = pl.ds(q * 16, 16)
                        rows[b, j, e, sl] = rows[b, j, e, sl] * w

        for j in range(SUBS):
            pltpu.async_copy(
                rows.at[b, j], acc.at[colb.at[islot, j]], ssem, add=True
            )
        return 0

    lax.fori_loop(0, NSUP, chunk, 0)
    lastb = (NSUP - 1) % 2
    lasts = (NSUP - 1) % 3
    for j in range(SUBS):
        pltpu.make_async_copy(
            rows.at[lastb, j], acc.at[colb.at[lasts, j]], ssem
        ).wait()
    plsc.subcore_barrier()
    pltpu.sync_copy(
        acc.at[pl.ds(sid * RPT, RPT)], out_hbm.at[cid, pl.ds(sid * RPT, RPT)]
    )


def _dinv_block(degpt):
    deg = degpt[:, 0:1] + degpt[:, 1:2] + 1.0
    return lax.rsqrt(deg)


def _mm1_body(x_ref, w_ref, degpt_ref, o_ref):
    dinv = _dinv_block(degpt_ref[...])
    h = jnp.dot(x_ref[...], w_ref[...], preferred_element_type=jnp.float32)
    o_ref[...] = h * dinv


def _layer_body(aggp_ref, hs_ref, degpt_ref, b_ref, w_ref, o_ref):
    dinv = _dinv_block(degpt_ref[...])
    s = aggp_ref[0] + aggp_ref[1] + hs_ref[...]
    act = jnp.maximum(s * dinv + b_ref[...], 0.0)
    o_ref[...] = (
        jnp.dot(act, w_ref[...], preferred_element_type=jnp.float32) * dinv
    )


def _pool_body(
    aggp_ref, hs_ref, degpt_ref, b_ref, batch_ref, wfc_ref, bfc_ref,
    o_ref, acc_ref, cnt_ref,
):
    i = pl.program_id(0)

    @pl.when(i == 0)
    def _():
        acc_ref[...] = jnp.zeros_like(acc_ref)
        cnt_ref[...] = jnp.zeros_like(cnt_ref)

    dinv = _dinv_block(degpt_ref[...])
    s = aggp_ref[0] + aggp_ref[1] + hs_ref[...]
    act = jnp.maximum(s * dinv + b_ref[...], 0.0)
    bvals = batch_ref[0]
    iot = lax.broadcasted_iota(jnp.int32, (G, MB), 0)
    oh = (iot == bvals).astype(jnp.float32)
    acc_ref[...] += jnp.dot(oh, act, preferred_element_type=jnp.float32)
    cnt_ref[...] += jnp.sum(oh, axis=1, keepdims=True)
    pooled = acc_ref[...] / jnp.maximum(cnt_ref[...], 1.0)
    o_ref[...] = (
        jnp.dot(pooled, wfc_ref[...], preferred_element_type=jnp.float32)
        + bfc_ref[...]
    )


_mm1 = pl.pallas_call(
    _mm1_body,
    grid=(NMB,),
    in_specs=[
        pl.BlockSpec((MB, FIN), lambda i: (i, 0)),
        pl.BlockSpec((FIN, H), lambda i: (0, 0)),
        pl.BlockSpec((MB, NC), lambda i: (i, 0)),
    ],
    out_specs=pl.BlockSpec((MB, H), lambda i: (i, 0)),
    out_shape=jax.ShapeDtypeStruct((NPAD, H), jnp.float32),
)

_layer = pl.pallas_call(
    _layer_body,
    grid=(NMB,),
    in_specs=[
        pl.BlockSpec((NC, MB, H), lambda i: (0, i, 0)),
        pl.BlockSpec((MB, H), lambda i: (i, 0)),
        pl.BlockSpec((MB, NC), lambda i: (i, 0)),
        pl.BlockSpec((1, H), lambda i: (0, 0)),
        pl.BlockSpec((H, H), lambda i: (0, 0)),
    ],
    out_specs=pl.BlockSpec((MB, H), lambda i: (i, 0)),
    out_shape=jax.ShapeDtypeStruct((NPAD, H), jnp.float32),
)

_pool = pl.pallas_call(
    _pool_body,
    grid=(NMB,),
    in_specs=[
        pl.BlockSpec((NC, MB, H), lambda i: (0, i, 0)),
        pl.BlockSpec((MB, H), lambda i: (i, 0)),
        pl.BlockSpec((MB, NC), lambda i: (i, 0)),
        pl.BlockSpec((1, H), lambda i: (0, 0)),
        pl.BlockSpec((1, 1, MB), lambda i: (i, 0, 0)),
        pl.BlockSpec((H, C), lambda i: (0, 0)),
        pl.BlockSpec((1, C), lambda i: (0, 0)),
    ],
    out_specs=pl.BlockSpec((G, C), lambda i: (0, 0)),
    out_shape=jax.ShapeDtypeStruct((G, C), jnp.float32),
    scratch_shapes=[
        pltpu.VMEM((G, H), jnp.float32),
        pltpu.VMEM((G, 1), jnp.float32),
    ],
)


def kernel(x, edge_index, edge_attr, batch, W1, b1, W2, b2, W3, b3, W4, b4,
           W5, b5, Wfc, bfc):
    row, col = edge_index[0], edge_index[1]
    row2d = row.reshape(E // CH, CH)
    col2d = col.reshape(E // CH, CH)
    ew2d = edge_attr.reshape(E // CH, CH)
    zeros = jnp.zeros((RPT, H), jnp.float32)
    _deg_kernel, _agg_kernel = _sc_kernels()
    degp = _deg_kernel(col, edge_attr)
    degpt = degp.T
    xp = jnp.concatenate(
        [x, jnp.zeros((NPAD - N, FIN), jnp.float32)], axis=0
    )
    bpad = jnp.concatenate(
        [batch, jnp.full((NPAD - N,), G, batch.dtype)]
    ).reshape(NMB, 1, MB)

    hs = _mm1(xp, W1, degpt)
    for b_prev, W_next in ((b1, W2), (b2, W3), (b3, W4), (b4, W5)):
        aggp = _agg_kernel(hs, row2d, col2d, ew2d, zeros)
        hs = _layer(aggp, hs, degpt, b_prev.reshape(1, H), W_next)
    aggp = _agg_kernel(hs, row2d, col2d, ew2d, zeros)
    return _pool(
        aggp, hs, degpt, b5.reshape(1, H), bpad, Wfc, bfc.reshape(1, C)
    )


# speedup vs baseline: 21.3829x; 1.0690x over previous
import functools

import jax
import jax.numpy as jnp
from jax import lax
from jax.experimental import pallas as pl
from jax.experimental.pallas import tpu as pltpu
from jax.experimental.pallas import tpu_sc as plsc

N, E, FIN, H, C, G = 10000, 320000, 128, 64, 10, 64
NC, NS = 2, 16
NW = NC * NS
EPW = E // NW
CH = 80
SUBS = 5
SUP = CH * SUBS
NSUP = EPW // SUP
NCHUNK = EPW // CH
NPAD = 10240
RPT = NPAD // NS
MB = 2048
NMB = NPAD // MB

@functools.cache
def _sc_kernels():
    mesh = plsc.VectorSubcoreMesh(
        core_axis_name="c", subcore_axis_name="s", num_cores=NC,
        num_subcores=NS,
    )
    params = pltpu.CompilerParams(use_tc_tiling_on_sc=False)
    deg = functools.partial(
        pl.kernel,
        out_type=jax.ShapeDtypeStruct((NC, NPAD), jnp.float32),
        mesh=mesh,
        compiler_params=params,
        scratch_types=[
            pltpu.VMEM((3, SUBS, CH), jnp.int32),
            pltpu.VMEM((3, SUBS, CH), jnp.float32),
            pltpu.VMEM((RPT,), jnp.float32),
            pltpu.VMEM_SHARED((NPAD,), jnp.float32),
            pltpu.SemaphoreType.DMA,
            pltpu.SemaphoreType.DMA,
            pltpu.SemaphoreType.DMA,
        ],
    )(_deg_body)
    agg = functools.partial(
        pl.kernel,
        out_type=jax.ShapeDtypeStruct((NC, NPAD, H), jnp.float32),
        mesh=mesh,
        compiler_params=params,
        scratch_types=[
            pltpu.VMEM((3, SUBS, CH), jnp.int32),
            pltpu.VMEM((3, SUBS, CH), jnp.int32),
            pltpu.VMEM((3, SUBS, CH), jnp.float32),
            pltpu.VMEM((3, CH, H), jnp.float32),
            pltpu.VMEM_SHARED((NPAD, H), jnp.float32),
            pltpu.SemaphoreType.DMA,
            pltpu.SemaphoreType.DMA,
            pltpu.SemaphoreType.DMA,
            pltpu.SemaphoreType.DMA,
        ],
    )(_agg_body)
    return deg, agg


def _deg_body(col_hbm, ew_hbm, out_hbm, colb, ewb, zbuf, acc,
              isem, ssem0, ssem1):
    cid = lax.axis_index("c")
    sid = lax.axis_index("s")
    wid = sid * NC + cid
    base = wid * (EPW // CH)

    def zb(i, _):
        zbuf[pl.ds(i * 16, 16)] = jnp.zeros((16,), jnp.float32)
        return 0

    lax.fori_loop(0, RPT // 16, zb, 0)
    pltpu.sync_copy(zbuf, acc.at[pl.ds(sid * RPT, RPT)])
    plsc.subcore_barrier()

    def issue_slab(t, slot):
        off = base + t * SUBS
        pltpu.async_copy(col_hbm.at[pl.ds(off, SUBS)], colb.at[slot], isem)
        pltpu.async_copy(ew_hbm.at[pl.ds(off, SUBS)], ewb.at[slot], isem)

    def drain_slab(slot):
        pltpu.make_async_copy(
            col_hbm.at[pl.ds(base, SUBS)], colb.at[slot], isem
        ).wait()
        pltpu.make_async_copy(
            ew_hbm.at[pl.ds(base, SUBS)], ewb.at[slot], isem
        ).wait()

    def drain_scatter(sem):
        pltpu.make_async_copy(
            ewb.at[0, 0], acc.at[colb.at[0, 0]], sem
        ).wait()

    pltpu.sync_copy(col_hbm.at[pl.ds(base, SUBS)], colb.at[0])
    pltpu.sync_copy(ew_hbm.at[pl.ds(base, SUBS)], ewb.at[0])
    issue_slab(1, 1)

    def chunk(k, _):
        s5 = lax.div(k, SUBS)
        j = k - s5 * SUBS
        islot = lax.rem(s5, 3)
        par = lax.rem(k, 2)
        @pl.when(k >= 2)
        def _():
            @pl.when(par == 0)
            def _():
                drain_scatter(ssem0)

            @pl.when(par == 1)
            def _():
                drain_scatter(ssem1)

        @pl.when(jnp.logical_and(j == 0, s5 + 1 <= NSUP - 1))
        def _():
            drain_slab(lax.rem(s5 + 1, 3))

        @pl.when(jnp.logical_and(j == 1, s5 + 2 <= NSUP - 1))
        def _():
            issue_slab(s5 + 2, lax.rem(s5 + 2, 3))

        @pl.when(par == 0)
        def _():
            pltpu.async_copy(
                ewb.at[islot, j], acc.at[colb.at[islot, j]], ssem0, add=True
            )

        @pl.when(par == 1)
        def _():
            pltpu.async_copy(
                ewb.at[islot, j], acc.at[colb.at[islot, j]], ssem1, add=True
            )

        return 0

    lax.fori_loop(0, NCHUNK, chunk, 0)
    drain_scatter(ssem1)
    drain_scatter(ssem0)
    plsc.subcore_barrier()
    pltpu.sync_copy(
        acc.at[pl.ds(sid * RPT, RPT)], out_hbm.at[cid, pl.ds(sid * RPT, RPT)]
    )


def _agg_body(
    hs_hbm, row_hbm, col_hbm, ew_hbm, zeros_hbm, out_hbm,
    rowb, colb, ewb, rows, acc, isem, gsem, ssem0, ssem1,
):
    cid = lax.axis_index("c")
    sid = lax.axis_index("s")
    wid = sid * NC + cid
    base = wid * (EPW // CH)

    pltpu.sync_copy(zeros_hbm, acc.at[pl.ds(sid * RPT, RPT)])
    plsc.subcore_barrier()

    def issue_slab(t, slot):
        off = base + t * SUBS
        pltpu.async_copy(row_hbm.at[pl.ds(off, SUBS)], rowb.at[slot], isem)
        pltpu.async_copy(col_hbm.at[pl.ds(off, SUBS)], colb.at[slot], isem)
        pltpu.async_copy(ew_hbm.at[pl.ds(off, SUBS)], ewb.at[slot], isem)

    def drain_slab(slot):
        pltpu.make_async_copy(
            row_hbm.at[pl.ds(base, SUBS)], rowb.at[slot], isem
        ).wait()
        pltpu.make_async_copy(
            col_hbm.at[pl.ds(base, SUBS)], colb.at[slot], isem
        ).wait()
        pltpu.make_async_copy(
            ew_hbm.at[pl.ds(base, SUBS)], ewb.at[slot], isem
        ).wait()

    def drain_scatter(sem):
        pltpu.make_async_copy(
            rows.at[0], acc.at[colb.at[0, 0]], sem
        ).wait()

    pltpu.sync_copy(row_hbm.at[pl.ds(base, SUBS)], rowb.at[0])
    pltpu.sync_copy(col_hbm.at[pl.ds(base, SUBS)], colb.at[0])
    pltpu.sync_copy(ew_hbm.at[pl.ds(base, SUBS)], ewb.at[0])
    issue_slab(1, 1)
    pltpu.async_copy(hs_hbm.at[rowb.at[0, 0]], rows.at[0], gsem)

    def chunk(k, _):
        s5 = lax.div(k, SUBS)
        j = k - s5 * SUBS
        islot = lax.rem(s5, 3)
        b = lax.rem(k, 3)
        par = lax.rem(k, 2)
        @pl.when(k >= 2)
        def _():
            @pl.when(par == 0)
            def _():
                drain_scatter(ssem0)

            @pl.when(par == 1)
            def _():
                drain_scatter(ssem1)

        @pl.when(jnp.logical_and(j == 0, s5 + 1 <= NSUP - 1))
        def _():
            drain_slab(lax.rem(s5 + 1, 3))

        @pl.when(jnp.logical_and(j == 1, s5 + 2 <= NSUP - 1))
        def _():
            issue_slab(s5 + 2, lax.rem(s5 + 2, 3))

        pltpu.make_async_copy(
            hs_hbm.at[rowb.at[islot, j]], rows.at[b], gsem
        ).wait()

        @pl.when(k < NCHUNK - 1)
        def _():
            k1 = k + 1
            s5n = lax.div(k1, SUBS)
            jn = k1 - s5n * SUBS
            pltpu.async_copy(
                hs_hbm.at[rowb.at[lax.rem(s5n, 3), jn]],
                rows.at[lax.rem(k1, 3)],
                gsem,
            )

        for g in range(CH // 16):
            ewv = ewb[islot, j, pl.ds(g * 16, 16)]
            for l in range(16):
                w = ewv.at[jnp.full((16,), l, jnp.int32)].get(
                    mode="promise_in_bounds"
                )
                e = g * 16 + l
                for q in range(H // 16):
                    sl = pl.ds(q * 16, 16)
                    rows[b, e, sl] = rows[b, e, sl] * w

        @pl.when(par == 0)
        def _():
            pltpu.async_copy(
                rows.at[b], acc.at[colb.at[islot, j]], ssem0, add=True
            )

        @pl.when(par == 1)
        def _():
            pltpu.async_copy(
                rows.at[b], acc.at[colb.at[islot, j]], ssem1, add=True
            )

        return 0

    lax.fori_loop(0, NCHUNK, chunk, 0)
    drain_scatter(ssem1)
    drain_scatter(ssem0)
    plsc.subcore_barrier()
    pltpu.sync_copy(
        acc.at[pl.ds(sid * RPT, RPT)], out_hbm.at[cid, pl.ds(sid * RPT, RPT)]
    )


def _dinv_block(degpt):
    deg = degpt[:, 0:1] + degpt[:, 1:2] + 1.0
    return lax.rsqrt(deg)


def _mm1_body(x_ref, w_ref, degpt_ref, o_ref):
    dinv = _dinv_block(degpt_ref[...])
    h = jnp.dot(x_ref[...], w_ref[...], preferred_element_type=jnp.float32)
    o_ref[...] = h * dinv


def _layer_body(aggp_ref, hs_ref, degpt_ref, b_ref, w_ref, o_ref):
    dinv = _dinv_block(degpt_ref[...])
    s = aggp_ref[0] + aggp_ref[1] + hs_ref[...]
    act = jnp.maximum(s * dinv + b_ref[...], 0.0)
    o_ref[...] = (
        jnp.dot(act, w_ref[...], preferred_element_type=jnp.float32) * dinv
    )


def _pool_body(
    aggp_ref, hs_ref, degpt_ref, b_ref, batch_ref, wfc_ref, bfc_ref,
    o_ref, acc_ref, cnt_ref,
):
    i = pl.program_id(0)

    @pl.when(i == 0)
    def _():
        acc_ref[...] = jnp.zeros_like(acc_ref)
        cnt_ref[...] = jnp.zeros_like(cnt_ref)

    dinv = _dinv_block(degpt_ref[...])
    s = aggp_ref[0] + aggp_ref[1] + hs_ref[...]
    act = jnp.maximum(s * dinv + b_ref[...], 0.0)
    bvals = batch_ref[0]
    iot = lax.broadcasted_iota(jnp.int32, (G, MB), 0)
    oh = (iot == bvals).astype(jnp.float32)
    acc_ref[...] += jnp.dot(oh, act, preferred_element_type=jnp.float32)
    cnt_ref[...] += jnp.sum(oh, axis=1, keepdims=True)
    pooled = acc_ref[...] / jnp.maximum(cnt_ref[...], 1.0)
    o_ref[...] = (
        jnp.dot(pooled, wfc_ref[...], preferred_element_type=jnp.float32)
        + bfc_ref[...]
    )


_mm1 = pl.pallas_call(
    _mm1_body,
    grid=(NMB,),
    in_specs=[
        pl.BlockSpec((MB, FIN), lambda i: (i, 0)),
        pl.BlockSpec((FIN, H), lambda i: (0, 0)),
        pl.BlockSpec((MB, NC), lambda i: (i, 0)),
    ],
    out_specs=pl.BlockSpec((MB, H), lambda i: (i, 0)),
    out_shape=jax.ShapeDtypeStruct((NPAD, H), jnp.float32),
)

_layer = pl.pallas_call(
    _layer_body,
    grid=(NMB,),
    in_specs=[
        pl.BlockSpec((NC, MB, H), lambda i: (0, i, 0)),
        pl.BlockSpec((MB, H), lambda i: (i, 0)),
        pl.BlockSpec((MB, NC), lambda i: (i, 0)),
        pl.BlockSpec((1, H), lambda i: (0, 0)),
        pl.BlockSpec((H, H), lambda i: (0, 0)),
    ],
    out_specs=pl.BlockSpec((MB, H), lambda i: (i, 0)),
    out_shape=jax.ShapeDtypeStruct((NPAD, H), jnp.float32),
)

_pool = pl.pallas_call(
    _pool_body,
    grid=(NMB,),
    in_specs=[
        pl.BlockSpec((NC, MB, H), lambda i: (0, i, 0)),
        pl.BlockSpec((MB, H), lambda i: (i, 0)),
        pl.BlockSpec((MB, NC), lambda i: (i, 0)),
        pl.BlockSpec((1, H), lambda i: (0, 0)),
        pl.BlockSpec((1, 1, MB), lambda i: (i, 0, 0)),
        pl.BlockSpec((H, C), lambda i: (0, 0)),
        pl.BlockSpec((1, C), lambda i: (0, 0)),
    ],
    out_specs=pl.BlockSpec((G, C), lambda i: (0, 0)),
    out_shape=jax.ShapeDtypeStruct((G, C), jnp.float32),
    scratch_shapes=[
        pltpu.VMEM((G, H), jnp.float32),
        pltpu.VMEM((G, 1), jnp.float32),
    ],
)


def kernel(x, edge_index, edge_attr, batch, W1, b1, W2, b2, W3, b3, W4, b4,
           W5, b5, Wfc, bfc):
    row, col = edge_index[0], edge_index[1]
    row2d = row.reshape(E // CH, CH)
    col2d = col.reshape(E // CH, CH)
    ew2d = edge_attr.reshape(E // CH, CH)
    zeros = jnp.zeros((RPT, H), jnp.float32)
    _deg_kernel, _agg_kernel = _sc_kernels()
    degp = _deg_kernel(col2d, ew2d)
    degpt = degp.T
    xp = jnp.concatenate(
        [x, jnp.zeros((NPAD - N, FIN), jnp.float32)], axis=0
    )
    bpad = jnp.concatenate(
        [batch, jnp.full((NPAD - N,), G, batch.dtype)]
    ).reshape(NMB, 1, MB)

    hs = _mm1(xp, W1, degpt)
    for b_prev, W_next in ((b1, W2), (b2, W3), (b3, W4), (b4, W5)):
        aggp = _agg_kernel(hs, row2d, col2d, ew2d, zeros)
        hs = _layer(aggp, hs, degpt, b_prev.reshape(1, H), W_next)
    aggp = _agg_kernel(hs, row2d, col2d, ew2d, zeros)
    return _pool(
        aggp, hs, degpt, b5.reshape(1, H), bpad, Wfc, bfc.reshape(1, C)
    )
